# baseline (speedup 1.0000x reference)
import functools
import math

import jax
import jax.numpy as jnp
from jax import lax
from jax.experimental import pallas as pl
from jax.experimental.pallas import tpu as pltpu
from jax.experimental.pallas import tpu_sc as plsc

D_MODEL = 64
VOCAB = 1000000
SCALE = math.sqrt(D_MODEL)
LANES = 16

NUM_CORES = 2
NUM_SUBCORES = 16
NUM_WORKERS = NUM_CORES * NUM_SUBCORES

PACK_LINES = VOCAB // 2

PACK_CHUNK = 384
PACK_J = PACK_CHUNK // LANES
PACK_FULL_CHUNKS = VOCAB // PACK_CHUNK
PACK_TAIL_START = PACK_FULL_CHUNKS * PACK_CHUNK
PACK_TAIL = VOCAB - PACK_TAIL_START
PACK_ITERS = -(-PACK_FULL_CHUNKS // NUM_WORKERS)

SEQ = 50
BATCH = 16384
CB = 128
N_BLOCKS = BATCH // CB
BLOCKS_PER_W = N_BLOCKS // NUM_WORKERS

_MESH = dict(core_axis_name="c", subcore_axis_name="s",
             num_cores=NUM_CORES, num_subcores=NUM_SUBCORES)


def _wid():
    return lax.axis_index("s") * NUM_CORES + lax.axis_index("c")


def _pack_kernel():
    mesh = plsc.VectorSubcoreMesh(**_MESH)

    @functools.partial(
        pl.kernel,
        out_type=jax.ShapeDtypeStruct((PACK_LINES, 128), jnp.float32),
        mesh=mesh,
        scratch_types=[
            pltpu.VMEM((D_MODEL, PACK_CHUNK + 1), jnp.float32),
            pltpu.VMEM((D_MODEL, PACK_CHUNK + 1), jnp.float32),
            pltpu.VMEM((PACK_CHUNK // 2, 128), jnp.float32),
            pltpu.VMEM((PACK_CHUNK // 2, 128), jnp.float32),
            pltpu.VMEM((PACK_TAIL // 2, 128), jnp.float32),
            pltpu.SemaphoreType.DMA,
            pltpu.SemaphoreType.DMA,
            pltpu.SemaphoreType.DMA,
            pltpu.SemaphoreType.DMA,
        ],
        compiler_params=pltpu.CompilerParams(use_tc_tiling_on_sc=True,
                                             needs_layout_passes=False),
    )
    def body(lutT_hbm, tail_hbm, packed_hbm, src0, src1, stg0, stg1, tail_v,
             ld0, ld1, st0, st1):
        w = _wid()
        iota = lax.iota(jnp.int32, LANES)
        zv = iota * 0

        def r_start(k, src, sem):
            g = lax.min(w + k * NUM_WORKERS, PACK_FULL_CHUNKS - 1)
            r0 = pl.multiple_of(g * PACK_CHUNK, PACK_CHUNK)
            pltpu.async_copy(lutT_hbm.at[:, pl.ds(r0, PACK_CHUNK)],
                             src.at[:, pl.ds(0, PACK_CHUNK)], sem)

        def r_wait(src, sem):
            pltpu.make_async_copy(lutT_hbm.at[:, pl.ds(0, PACK_CHUNK)],
                                  src.at[:, pl.ds(0, PACK_CHUNK)], sem).wait()

        def w_start(k, stg, sem):
            g = lax.min(w + k * NUM_WORKERS, PACK_FULL_CHUNKS - 1)
            l0 = pl.multiple_of(g * (PACK_CHUNK // 2), PACK_CHUNK // 2)
            pltpu.async_copy(stg, packed_hbm.at[pl.ds(l0, PACK_CHUNK // 2)],
                             sem)

        def w_wait(stg, sem):
            pltpu.make_async_copy(stg,
                                  packed_hbm.at[pl.ds(0, PACK_CHUNK // 2)],
                                  sem).wait()

        def rearrange(src_ref, stg_ref, n_lines):
            def l_body(l):
                for h in range(2):
                    rv = zv + (2 * l + h)
                    for cg in range(4):
                        vals = plsc.load_gather(
                            src_ref, [iota + cg * LANES, rv])
                        stg_ref[l, pl.ds(h * D_MODEL + cg * LANES,
                                         LANES)] = vals
            plsc.parallel_loop(0, n_lines, 1, unroll=2)(l_body)

        r_start(0, src0, ld0)
        r_start(1, src1, ld1)

        def t_loop(t, _):
            k0 = 2 * t
            r_wait(src0, ld0)

            @pl.when(t > 0)
            def _():
                w_wait(stg0, st0)
            rearrange(src0, stg0, PACK_CHUNK // 2)
            w_start(k0, stg0, st0)

            @pl.when(k0 + 2 < PACK_ITERS)
            def _():
                r_start(k0 + 2, src0, ld0)

            r_wait(src1, ld1)

            @pl.when(t > 0)
            def _():
                w_wait(stg1, st1)
            rearrange(src1, stg1, PACK_CHUNK // 2)
            w_start(k0 + 1, stg1, st1)

            @pl.when(k0 + 3 < PACK_ITERS)
            def _():
                r_start(k0 + 3, src1, ld1)
            return 0

        lax.fori_loop(0, PACK_ITERS // 2, t_loop, 0)
        w_wait(stg0, st0)
        w_wait(stg1, st1)

        @pl.when(w == NUM_WORKERS - 1)
        def _():
            pltpu.sync_copy(tail_hbm, tail_v)
            pltpu.sync_copy(tail_v,
                            packed_hbm.at[pl.ds(PACK_TAIL_START // 2,
                                                PACK_TAIL // 2)])

    return body


def _gather_kernel():
    mesh = plsc.VectorSubcoreMesh(**_MESH)

    @functools.partial(
        pl.kernel,
        out_type=jax.ShapeDtypeStruct((SEQ, 8, BATCH // 128, 8, 128),
                                      jnp.float32),
        mesh=mesh,
        scratch_types=[
            pltpu.VMEM((CB * SEQ,), jnp.int32),
            pltpu.VMEM((CB // 128, 128), jnp.int32),
            pltpu.VMEM((CB // 128, 128), jnp.int32),
            pltpu.VMEM((CB,), jnp.int32),
            pltpu.VMEM((CB,), jnp.int32),
            pltpu.VMEM((CB, 129), jnp.float32),
            pltpu.VMEM((CB, 129), jnp.float32),
            pltpu.VMEM((8, CB // 128, 8, 128), jnp.float32),
            pltpu.SemaphoreType.DMA,
            pltpu.SemaphoreType.DMA,
        ],
        compiler_params=pltpu.CompilerParams(use_tc_tiling_on_sc=True,
                                             needs_layout_passes=False),
    )
    def body(xf_hbm, packed_hbm, out_hbm, xv, idx0, idx1, h0, h1,
             rows0, rows1, stg, g0, g1):
        w = _wid()
        iota = lax.iota(jnp.int32, LANES)
        iota_seq = iota * SEQ

        def extract_and_fire(s, b0, idx_v, h_v, rows_v, sem):
            for bg in range(CB // LANES):
                offs = iota_seq + (bg * LANES * SEQ + s)
                v = plsc.load_gather(xv, [offs])
                idx_v[bg // 8, pl.ds((bg % 8) * LANES, LANES)] = (
                    lax.shift_right_logical(v, 1))
                h_v[pl.ds(bg * LANES, LANES)] = (
                    lax.bitwise_and(v, 1) * D_MODEL)
            for q in range(CB // 128):
                pltpu.async_copy(
                    packed_hbm.at[idx_v.at[q]],
                    rows_v.at[pl.ds(q * 128, 128), pl.ds(0, 128)], sem)

        def g_wait(idx_v, rows_v, sem):
            for q in range(CB // 128):
                pltpu.make_async_copy(
                    packed_hbm.at[idx_v.at[q]],
                    rows_v.at[pl.ds(q * 128, 128), pl.ds(0, 128)],
                    sem).wait()

        def rearrange(rows_v, h_v, stg_v):
            for bg in range(CB // LANES):
                bidx = iota + bg * LANES
                bb = bg // 8
                bc0 = (bg % 8) * LANES
                hv = h_v[pl.ds(bg * LANES, LANES)]

                def d_body(d):
                    vals = plsc.load_gather(rows_v, [bidx, hv + d])
                    d8 = lax.shift_right_logical(d, 3)
                    dr = lax.bitwise_and(d, 7)
                    stg_v[d8, bb, dr, pl.ds(bc0, LANES)] = vals * SCALE
                plsc.parallel_loop(0, D_MODEL, 1, unroll=4)(d_body)

        def block_body(bi, _):
            blk = w * BLOCKS_PER_W + bi
            b0 = pl.multiple_of(blk * CB, CB)
            pltpu.sync_copy(xf_hbm.at[pl.ds(b0 * SEQ, CB * SEQ)], xv)

            extract_and_fire(0, b0, idx0, h0, rows0, g0)
            extract_and_fire(1, b0, idx1, h1, rows1, g1)

            def s_loop(s, _):
                even = lax.bitwise_and(s, 1) == 0

                @pl.when(even)
                def _():
                    g_wait(idx0, rows0, g0)
                    rearrange(rows0, h0, stg)

                @pl.when(jnp.logical_not(even))
                def _():
                    g_wait(idx1, rows1, g1)
                    rearrange(rows1, h1, stg)

                bb0 = pl.multiple_of(b0 // 128, CB // 128)
                pltpu.sync_copy(
                    stg, out_hbm.at[s, :, pl.ds(bb0, CB // 128), :, :])

                @pl.when(jnp.logical_and(even, s + 2 < SEQ))
                def _():
                    extract_and_fire(s + 2, b0, idx0, h0, rows0, g0)

                @pl.when(jnp.logical_and(jnp.logical_not(even), s + 2 < SEQ))
                def _():
                    extract_and_fire(s + 2, b0, idx1, h1, rows1, g1)
                return 0

            lax.fori_loop(0, SEQ, s_loop, 0)
            return 0

        lax.fori_loop(0, BLOCKS_PER_W, block_body, 0)

    return body


def kernel(x, lut):
    xf = x.reshape(-1).astype(jnp.int32)
    tail = lut[PACK_TAIL_START:].reshape(PACK_TAIL // 2, 128)
    packed = _pack_kernel()(lut.T, tail)
    out5 = _gather_kernel()(xf, packed)
    return jnp.transpose(out5, (2, 4, 0, 1, 3)).reshape(BATCH, SEQ, D_MODEL)


# speedup vs baseline: 2.7156x; 2.7156x over previous
import functools
import math

import jax
import jax.numpy as jnp
from jax import lax
from jax.experimental import pallas as pl
from jax.experimental.pallas import tpu as pltpu
from jax.experimental.pallas import tpu_sc as plsc

D_MODEL = 64
VOCAB = 1000000
SCALE = math.sqrt(D_MODEL)
LANES = 16

NUM_CORES = 2
NUM_SUBCORES = 16
NUM_WORKERS = NUM_CORES * NUM_SUBCORES

PACK_LINES = VOCAB // 2

PACK_CHUNK = 384
PACK_J = PACK_CHUNK // LANES
PACK_FULL_CHUNKS = VOCAB // PACK_CHUNK
PACK_TAIL_START = PACK_FULL_CHUNKS * PACK_CHUNK
PACK_TAIL = VOCAB - PACK_TAIL_START
PACK_ITERS = -(-PACK_FULL_CHUNKS // NUM_WORKERS)

SEQ = 50
BATCH = 16384
CB = 256
N_BLOCKS = BATCH // CB
BLOCKS_PER_W = N_BLOCKS // NUM_WORKERS

_MESH = dict(core_axis_name="c", subcore_axis_name="s",
             num_cores=NUM_CORES, num_subcores=NUM_SUBCORES)


def _wid():
    return lax.axis_index("s") * NUM_CORES + lax.axis_index("c")


def _pack_kernel():
    mesh = plsc.VectorSubcoreMesh(**_MESH)

    @functools.partial(
        pl.kernel,
        out_type=jax.ShapeDtypeStruct((PACK_LINES, 128), jnp.float32),
        mesh=mesh,
        scratch_types=[
            pltpu.VMEM((D_MODEL, PACK_CHUNK + 1), jnp.float32),
            pltpu.VMEM((D_MODEL, PACK_CHUNK + 1), jnp.float32),
            pltpu.VMEM((PACK_CHUNK // 2, 128), jnp.float32),
            pltpu.VMEM((PACK_CHUNK // 2, 128), jnp.float32),
            pltpu.VMEM((PACK_TAIL // 2, 128), jnp.float32),
            pltpu.SemaphoreType.DMA,
            pltpu.SemaphoreType.DMA,
            pltpu.SemaphoreType.DMA,
            pltpu.SemaphoreType.DMA,
        ],
        compiler_params=pltpu.CompilerParams(use_tc_tiling_on_sc=True,
                                             needs_layout_passes=False),
    )
    def body(lutT_hbm, tail_hbm, packed_hbm, src0, src1, stg0, stg1, tail_v,
             ld0, ld1, st0, st1):
        w = _wid()
        iota = lax.iota(jnp.int32, LANES)
        zv = iota * 0

        def r_start(k, src, sem):
            g = lax.min(w + k * NUM_WORKERS, PACK_FULL_CHUNKS - 1)
            r0 = pl.multiple_of(g * PACK_CHUNK, PACK_CHUNK)
            pltpu.async_copy(lutT_hbm.at[:, pl.ds(r0, PACK_CHUNK)],
                             src.at[:, pl.ds(0, PACK_CHUNK)], sem)

        def r_wait(src, sem):
            pltpu.make_async_copy(lutT_hbm.at[:, pl.ds(0, PACK_CHUNK)],
                                  src.at[:, pl.ds(0, PACK_CHUNK)], sem).wait()

        def w_start(k, stg, sem):
            g = lax.min(w + k * NUM_WORKERS, PACK_FULL_CHUNKS - 1)
            l0 = pl.multiple_of(g * (PACK_CHUNK // 2), PACK_CHUNK // 2)
            pltpu.async_copy(stg, packed_hbm.at[pl.ds(l0, PACK_CHUNK // 2)],
                             sem)

        def w_wait(stg, sem):
            pltpu.make_async_copy(stg,
                                  packed_hbm.at[pl.ds(0, PACK_CHUNK // 2)],
                                  sem).wait()

        def rearrange(src_ref, stg_ref, n_lines):
            def r_body(r0):
                rv = r0 * 2 + iota
                lv = lax.shift_right_logical(rv, 1)
                hv = lax.bitwise_and(rv, 1) * D_MODEL
                for cg in range(4):
                    cv = lax.rem(iota + cg * LANES + r0, zv + D_MODEL)
                    vals = plsc.load_gather(src_ref, [cv, rv])
                    plsc.store_scatter(stg_ref, [lv, hv + cv], vals)
            plsc.parallel_loop(0, n_lines, 1, unroll=2)(r_body)

        r_start(0, src0, ld0)
        r_start(1, src1, ld1)

        def t_loop(t, _):
            k0 = 2 * t
            r_wait(src0, ld0)

            @pl.when(t > 0)
            def _():
                w_wait(stg0, st0)
            rearrange(src0, stg0, PACK_CHUNK // 2)
            w_start(k0, stg0, st0)

            @pl.when(k0 + 2 < PACK_ITERS)
            def _():
                r_start(k0 + 2, src0, ld0)

            r_wait(src1, ld1)

            @pl.when(t > 0)
            def _():
                w_wait(stg1, st1)
            rearrange(src1, stg1, PACK_CHUNK // 2)
            w_start(k0 + 1, stg1, st1)

            @pl.when(k0 + 3 < PACK_ITERS)
            def _():
                r_start(k0 + 3, src1, ld1)
            return 0

        lax.fori_loop(0, PACK_ITERS // 2, t_loop, 0)
        w_wait(stg0, st0)
        w_wait(stg1, st1)

        @pl.when(w == NUM_WORKERS - 1)
        def _():
            pltpu.sync_copy(tail_hbm, tail_v)
            pltpu.sync_copy(tail_v,
                            packed_hbm.at[pl.ds(PACK_TAIL_START // 2,
                                                PACK_TAIL // 2)])

    return body


def _gather_kernel():
    mesh = plsc.VectorSubcoreMesh(**_MESH)

    @functools.partial(
        pl.kernel,
        out_type=jax.ShapeDtypeStruct((SEQ, 8, BATCH // 128, 8, 128),
                                      jnp.float32),
        mesh=mesh,
        scratch_types=[
            pltpu.VMEM((CB * SEQ,), jnp.int32),
            pltpu.VMEM((CB // 128, 128), jnp.int32),
            pltpu.VMEM((CB // 128, 128), jnp.int32),
            pltpu.VMEM((CB,), jnp.int32),
            pltpu.VMEM((CB,), jnp.int32),
            pltpu.VMEM((CB, 129), jnp.float32),
            pltpu.VMEM((CB, 129), jnp.float32),
            pltpu.VMEM((8, CB // 128, 8, 128), jnp.float32),
            pltpu.SemaphoreType.DMA,
            pltpu.SemaphoreType.DMA,
        ],
        compiler_params=pltpu.CompilerParams(use_tc_tiling_on_sc=False,
                                             needs_layout_passes=False),
    )
    def body(xf_hbm, packed_hbm, out_hbm, xv, idx0, idx1, h0, h1,
             rows0, rows1, stg, g0, g1):
        w = _wid()
        iota = lax.iota(jnp.int32, LANES)
        iota_seq = iota * SEQ

        def extract_and_fire(s, b0, idx_v, h_v, rows_v, sem):
            for bg in range(CB // LANES):
                offs = iota_seq + (bg * LANES * SEQ + s)
                v = plsc.load_gather(xv, [offs])
                idx_v[bg // 8, pl.ds((bg % 8) * LANES, LANES)] = v
            for q in range(CB // 128):
                pltpu.async_copy(
                    packed_hbm.at[idx_v.at[q]],
                    rows_v.at[pl.ds(q * 128, 128), pl.ds(0, 128)], sem)

        def g_wait(idx_v, rows_v, sem):
            for q in range(CB // 128):
                pltpu.make_async_copy(
                    packed_hbm.at[idx_v.at[q]],
                    rows_v.at[pl.ds(q * 128, 128), pl.ds(0, 128)],
                    sem).wait()

        def rearrange(rows_v, h_v, stg_v):
            for bg in range(CB // LANES):
                bidx = iota + bg * LANES
                bbv = iota * 0 + (bg // 8)
                bcv = iota + (bg % 8) * LANES

                def d_body(d0):
                    dv = lax.bitwise_and(d0 + iota, D_MODEL - 1)
                    vals = plsc.load_gather(rows_v, [bidx, dv])
                    d8v = lax.shift_right_logical(dv, 3)
                    drv = lax.bitwise_and(dv, 7)
                    plsc.store_scatter(stg_v, [d8v, bbv, drv, bcv],
                                       vals * SCALE)
                plsc.parallel_loop(0, D_MODEL, 1, unroll=8)(d_body)

        def block_body(bi, _):
            blk = w * BLOCKS_PER_W + bi
            b0 = pl.multiple_of(blk * CB, CB)
            pltpu.sync_copy(xf_hbm.at[pl.ds(b0 * SEQ, CB * SEQ)], xv)

            extract_and_fire(0, b0, idx0, h0, rows0, g0)
            extract_and_fire(1, b0, idx1, h1, rows1, g1)

            def s_loop(s, _):
                even = lax.bitwise_and(s, 1) == 0

                @pl.when(even)
                def _():
                    g_wait(idx0, rows0, g0)
                    rearrange(rows0, h0, stg)

                @pl.when(jnp.logical_not(even))
                def _():
                    g_wait(idx1, rows1, g1)
                    rearrange(rows1, h1, stg)

                bb0 = pl.multiple_of(b0 // 128, CB // 128)
                pltpu.sync_copy(
                    stg, out_hbm.at[s, :, pl.ds(bb0, CB // 128), :, :])

                @pl.when(jnp.logical_and(even, s + 2 < SEQ))
                def _():
                    extract_and_fire(s + 2, b0, idx0, h0, rows0, g0)

                @pl.when(jnp.logical_and(jnp.logical_not(even), s + 2 < SEQ))
                def _():
                    extract_and_fire(s + 2, b0, idx1, h1, rows1, g1)
                return 0

            lax.fori_loop(0, SEQ, s_loop, 0)
            return 0

        lax.fori_loop(0, BLOCKS_PER_W, block_body, 0)

    return body


def kernel(x, lut):
    xf = x.reshape(-1).astype(jnp.int32)
    tail = lut[PACK_TAIL_START:].reshape(PACK_TAIL // 2, 128)
    packed = _pack_kernel()(lut.T, tail)
    out5 = _gather_kernel()(xf, packed.reshape(VOCAB, D_MODEL))
    return jnp.transpose(out5, (2, 4, 0, 1, 3)).reshape(BATCH, SEQ, D_MODEL)
